# baseline (speedup 1.0000x reference)
import jax
import jax.numpy as jnp
from jax.experimental import pallas as pl
from jax.experimental.pallas import tpu as pltpu

_Q = 4096
_D = 128
_K = 100000
_QT = 256
_KT = 2048
_KTILES = 50
_KPAD = _KT * _KTILES
_TOPK = 16
_NEG = float("-inf")


def _knn_kernel(q_ref, k_ref, ksq_ref, sc_ref, ix_ref, nd_ref, rv_ref, ri_ref):
    j = pl.program_id(1)

    @pl.when(j == 0)
    def _init():
        rv_ref[...] = jnp.full((_QT, _TOPK), _NEG, jnp.float32)
        ri_ref[...] = jnp.zeros((_QT, _TOPK), jnp.float32)

    q = q_ref[...]
    kt = k_ref[...]
    dots = jax.lax.dot_general(
        q, kt,
        dimension_numbers=(((1,), (1,)), ((), ())),
        preferred_element_type=jnp.float32,
        precision=jax.lax.Precision.DEFAULT,
    )
    qsq = jnp.sum(q * q, axis=1, keepdims=True)
    nd_ref[...] = -((qsq - 2.0 * dots) + ksq_ref[...])

    si = jax.lax.broadcasted_iota(jnp.int32, (_QT, _TOPK), 1).astype(jnp.float32)
    _H = _KT // 2
    lih = jax.lax.broadcasted_iota(jnp.int32, (_QT, _H), 1).astype(jnp.float32)
    jbase = (j * _KT).astype(jnp.float32)

    t0 = jnp.min(rv_ref[...], axis=1, keepdims=True)
    cntl = jnp.sum((nd_ref[:, :_H] > t0).astype(jnp.int32), axis=1)
    cntr = jnp.sum((nd_ref[:, _H:] > t0).astype(jnp.int32), axis=1)
    rounds = jnp.minimum(jnp.max(jnp.maximum(cntl, cntr)), _TOPK)

    def _round(r, carry):
        del r
        subl = nd_ref[:, :_H]
        subr = nd_ref[:, _H:]
        ml = jnp.max(subl, axis=1, keepdims=True)
        mr = jnp.max(subr, axis=1, keepdims=True)
        posl = jnp.min(jnp.where(subl == ml, lih, float(_H)),
                       axis=1, keepdims=True)
        posr = jnp.min(jnp.where(subr == mr, lih, float(_H)),
                       axis=1, keepdims=True)
        nd_ref[:, :_H] = jnp.where(lih == posl, _NEG, subl)
        nd_ref[:, _H:] = jnp.where(lih == posr, _NEG, subr)
        rv = rv_ref[...]
        ri = ri_ref[...]
        t = jnp.min(rv, axis=1, keepdims=True)
        updl = ml > t
        slotl = jnp.min(jnp.where(rv == t, si, float(_TOPK)),
                        axis=1, keepdims=True)
        sell = (si == slotl) & updl
        rv1 = jnp.where(sell, ml, rv)
        ri1 = jnp.where(sell, jbase + posl, ri)
        t2 = jnp.min(rv1, axis=1, keepdims=True)
        updr = mr > t2
        slotr = jnp.min(jnp.where(rv1 == t2, si, float(_TOPK)),
                        axis=1, keepdims=True)
        selr = (si == slotr) & updr
        rv_ref[...] = jnp.where(selr, mr, rv1)
        ri_ref[...] = jnp.where(selr, jbase + float(_H) + posr, ri1)
        return carry

    jax.lax.fori_loop(0, rounds, _round, 0)

    @pl.when(j == _KTILES - 1)
    def _emit():
        vals = rv_ref[...]
        idxs = ri_ref[...]
        out_v = jnp.zeros((_QT, _TOPK), jnp.float32)
        out_i = jnp.zeros((_QT, _TOPK), jnp.float32)
        for r in range(_TOPK):
            m = jnp.max(vals, axis=1, keepdims=True)
            gi = jnp.min(jnp.where(vals == m, idxs, jnp.float32(2.0**25)),
                         axis=1, keepdims=True)
            kill = jnp.min(jnp.where((vals == m) & (idxs == gi), si,
                                     float(_TOPK)),
                           axis=1, keepdims=True)
            vals = jnp.where(si == kill, _NEG, vals)
            out_v = jnp.where(si == float(r), m, out_v)
            out_i = jnp.where(si == float(r), gi, out_i)
        sc_ref[...] = out_v
        ix_ref[...] = out_i.astype(jnp.int32)


def kernel(queries, keys, k):
    del k
    keys_p = jnp.concatenate(
        [keys, jnp.zeros((_KPAD - _K, _D), dtype=keys.dtype)], axis=0)
    ksq = jnp.sum(keys * keys, axis=1)
    ksq_p = jnp.concatenate(
        [ksq, jnp.full((_KPAD - _K,), jnp.inf, jnp.float32)])[None, :]
    scores, idx = pl.pallas_call(
        _knn_kernel,
        grid=(_Q // _QT, _KTILES),
        in_specs=[
            pl.BlockSpec((_QT, _D), lambda i, j: (i, 0)),
            pl.BlockSpec((_KT, _D), lambda i, j: (j, 0)),
            pl.BlockSpec((1, _KT), lambda i, j: (0, j)),
        ],
        out_specs=[
            pl.BlockSpec((_QT, _TOPK), lambda i, j: (i, 0)),
            pl.BlockSpec((_QT, _TOPK), lambda i, j: (i, 0)),
        ],
        out_shape=[
            jax.ShapeDtypeStruct((_Q, _TOPK), jnp.float32),
            jax.ShapeDtypeStruct((_Q, _TOPK), jnp.int32),
        ],
        scratch_shapes=[
            pltpu.VMEM((_QT, _KT), jnp.float32),
            pltpu.VMEM((_QT, _TOPK), jnp.float32),
            pltpu.VMEM((_QT, _TOPK), jnp.float32),
        ],
    )(queries, keys_p, ksq_p)
    return scores, idx


# speedup vs baseline: 1.0513x; 1.0513x over previous
import jax
import jax.numpy as jnp
from jax.experimental import pallas as pl
from jax.experimental.pallas import tpu as pltpu

_Q = 4096
_D = 128
_K = 100000
_QT = 256
_KT = 2048
_KTILES = 50
_KPAD = _KT * _KTILES
_TOPK = 16
_NEG = float("-inf")


def _knn_kernel(q_ref, k_ref, ksq_ref, sc_ref, ix_ref, nd_ref, rv_ref, ri_ref):
    j = pl.program_id(1)

    @pl.when(j == 0)
    def _init():
        rv_ref[...] = jnp.full((_QT, _TOPK), _NEG, jnp.float32)
        ri_ref[...] = jnp.zeros((_QT, _TOPK), jnp.float32)

    q = q_ref[...]
    kt = k_ref[...]
    dots = jax.lax.dot_general(
        q, kt,
        dimension_numbers=(((1,), (1,)), ((), ())),
        preferred_element_type=jnp.float32,
        precision=jax.lax.Precision.DEFAULT,
    )
    qsq = jnp.sum(q * q, axis=1, keepdims=True)
    nd_ref[...] = -((qsq - 2.0 * dots) + ksq_ref[...])

    si = jax.lax.broadcasted_iota(jnp.int32, (_QT, _TOPK), 1).astype(jnp.float32)
    _H = _KT // 2
    lih = jax.lax.broadcasted_iota(jnp.int32, (_QT, _H), 1).astype(jnp.float32)
    jbase = (j * _KT).astype(jnp.float32)

    t0 = jnp.min(rv_ref[...], axis=1, keepdims=True)
    cntl = jnp.sum((nd_ref[:, :_H] > t0).astype(jnp.int32), axis=1)
    cntr = jnp.sum((nd_ref[:, _H:] > t0).astype(jnp.int32), axis=1)
    roundsl = jnp.minimum(jnp.max(cntl), _TOPK)
    roundsr = jnp.minimum(jnp.max(cntr), _TOPK)

    def _half_round(base, sl):
        def _round(r, carry):
            del r
            sub = nd_ref[:, sl]
            m = jnp.max(sub, axis=1, keepdims=True)
            pos = jnp.min(jnp.where(sub == m, lih, float(_H)),
                          axis=1, keepdims=True)
            nd_ref[:, sl] = jnp.where(lih == pos, _NEG, sub)
            rv = rv_ref[...]
            t = jnp.min(rv, axis=1, keepdims=True)
            upd = m > t
            slot = jnp.min(jnp.where(rv == t, si, float(_TOPK)),
                           axis=1, keepdims=True)
            sel = (si == slot) & upd
            rv_ref[...] = jnp.where(sel, m, rv)
            ri_ref[...] = jnp.where(sel, base + pos, ri_ref[...])
            return carry
        return _round

    jax.lax.fori_loop(0, roundsl, _half_round(jbase, pl.ds(0, _H)), 0)
    jax.lax.fori_loop(0, roundsr,
                      _half_round(jbase + float(_H), pl.ds(_H, _H)), 0)

    @pl.when(j == _KTILES - 1)
    def _emit():
        vals = rv_ref[...]
        idxs = ri_ref[...]
        out_v = jnp.zeros((_QT, _TOPK), jnp.float32)
        out_i = jnp.zeros((_QT, _TOPK), jnp.float32)
        for r in range(_TOPK):
            m = jnp.max(vals, axis=1, keepdims=True)
            gi = jnp.min(jnp.where(vals == m, idxs, jnp.float32(2.0**25)),
                         axis=1, keepdims=True)
            kill = jnp.min(jnp.where((vals == m) & (idxs == gi), si,
                                     float(_TOPK)),
                           axis=1, keepdims=True)
            vals = jnp.where(si == kill, _NEG, vals)
            out_v = jnp.where(si == float(r), m, out_v)
            out_i = jnp.where(si == float(r), gi, out_i)
        sc_ref[...] = out_v
        ix_ref[...] = out_i.astype(jnp.int32)


def kernel(queries, keys, k):
    del k
    keys_p = jnp.concatenate(
        [keys, jnp.zeros((_KPAD - _K, _D), dtype=keys.dtype)], axis=0)
    ksq = jnp.sum(keys * keys, axis=1)
    ksq_p = jnp.concatenate(
        [ksq, jnp.full((_KPAD - _K,), jnp.inf, jnp.float32)])[None, :]
    scores, idx = pl.pallas_call(
        _knn_kernel,
        grid=(_Q // _QT, _KTILES),
        in_specs=[
            pl.BlockSpec((_QT, _D), lambda i, j: (i, 0)),
            pl.BlockSpec((_KT, _D), lambda i, j: (j, 0)),
            pl.BlockSpec((1, _KT), lambda i, j: (0, j)),
        ],
        out_specs=[
            pl.BlockSpec((_QT, _TOPK), lambda i, j: (i, 0)),
            pl.BlockSpec((_QT, _TOPK), lambda i, j: (i, 0)),
        ],
        out_shape=[
            jax.ShapeDtypeStruct((_Q, _TOPK), jnp.float32),
            jax.ShapeDtypeStruct((_Q, _TOPK), jnp.int32),
        ],
        scratch_shapes=[
            pltpu.VMEM((_QT, _KT), jnp.float32),
            pltpu.VMEM((_QT, _TOPK), jnp.float32),
            pltpu.VMEM((_QT, _TOPK), jnp.float32),
        ],
    )(queries, keys_p, ksq_p)
    return scores, idx


# speedup vs baseline: 1.0835x; 1.0306x over previous
import jax
import jax.numpy as jnp
from jax.experimental import pallas as pl
from jax.experimental.pallas import tpu as pltpu

_Q = 4096
_D = 128
_K = 100000
_QT = 256
_KT = 4096
_KTILES = 25
_NSPLIT = 4
_KPAD = _KT * _KTILES
_TOPK = 16
_NEG = float("-inf")


def _knn_kernel(q_ref, k_ref, ksq_ref, sc_ref, ix_ref, nd_ref, rv_ref, ri_ref):
    j = pl.program_id(1)

    @pl.when(j == 0)
    def _init():
        rv_ref[...] = jnp.full((_QT, _TOPK), _NEG, jnp.float32)
        ri_ref[...] = jnp.zeros((_QT, _TOPK), jnp.float32)

    q = q_ref[...]
    kt = k_ref[...]
    dots = jax.lax.dot_general(
        q, kt,
        dimension_numbers=(((1,), (1,)), ((), ())),
        preferred_element_type=jnp.float32,
        precision=jax.lax.Precision.DEFAULT,
    )
    qsq = jnp.sum(q * q, axis=1, keepdims=True)
    nd_ref[...] = -((qsq - 2.0 * dots) + ksq_ref[...])

    si = jax.lax.broadcasted_iota(jnp.int32, (_QT, _TOPK), 1).astype(jnp.float32)
    _SW = _KT // _NSPLIT
    lih = jax.lax.broadcasted_iota(jnp.int32, (_QT, _SW), 1).astype(jnp.float32)
    jbase = (j * _KT).astype(jnp.float32)

    t0 = jnp.min(rv_ref[...], axis=1, keepdims=True)
    rounds = []
    for p in range(_NSPLIT):
        cnt = jnp.sum((nd_ref[:, p * _SW:(p + 1) * _SW] > t0).astype(jnp.int32),
                      axis=1)
        rounds.append(jnp.minimum(jnp.max(cnt), _TOPK))

    def _split_round(base, sl):
        def _round(r, carry):
            del r
            sub = nd_ref[:, sl]
            m = jnp.max(sub, axis=1, keepdims=True)
            pos = jnp.min(jnp.where(sub == m, lih, float(_SW)),
                          axis=1, keepdims=True)
            nd_ref[:, sl] = jnp.where(lih == pos, _NEG, sub)
            rv = rv_ref[...]
            t = jnp.min(rv, axis=1, keepdims=True)
            upd = m > t
            slot = jnp.min(jnp.where(rv == t, si, float(_TOPK)),
                           axis=1, keepdims=True)
            sel = (si == slot) & upd
            rv_ref[...] = jnp.where(sel, m, rv)
            ri_ref[...] = jnp.where(sel, base + pos, ri_ref[...])
            return carry
        return _round

    for p in range(_NSPLIT):
        jax.lax.fori_loop(
            0, rounds[p],
            _split_round(jbase + float(p * _SW), pl.ds(p * _SW, _SW)), 0)

    @pl.when(j == _KTILES - 1)
    def _emit():
        vals = rv_ref[...]
        idxs = ri_ref[...]
        out_v = jnp.zeros((_QT, _TOPK), jnp.float32)
        out_i = jnp.zeros((_QT, _TOPK), jnp.float32)
        for r in range(_TOPK):
            m = jnp.max(vals, axis=1, keepdims=True)
            gi = jnp.min(jnp.where(vals == m, idxs, jnp.float32(2.0**25)),
                         axis=1, keepdims=True)
            kill = jnp.min(jnp.where((vals == m) & (idxs == gi), si,
                                     float(_TOPK)),
                           axis=1, keepdims=True)
            vals = jnp.where(si == kill, _NEG, vals)
            out_v = jnp.where(si == float(r), m, out_v)
            out_i = jnp.where(si == float(r), gi, out_i)
        sc_ref[...] = out_v
        ix_ref[...] = out_i.astype(jnp.int32)


def kernel(queries, keys, k):
    del k
    keys_p = jnp.concatenate(
        [keys, jnp.zeros((_KPAD - _K, _D), dtype=keys.dtype)], axis=0)
    ksq = jnp.sum(keys * keys, axis=1)
    ksq_p = jnp.concatenate(
        [ksq, jnp.full((_KPAD - _K,), jnp.inf, jnp.float32)])[None, :]
    scores, idx = pl.pallas_call(
        _knn_kernel,
        grid=(_Q // _QT, _KTILES),
        in_specs=[
            pl.BlockSpec((_QT, _D), lambda i, j: (i, 0)),
            pl.BlockSpec((_KT, _D), lambda i, j: (j, 0)),
            pl.BlockSpec((1, _KT), lambda i, j: (0, j)),
        ],
        out_specs=[
            pl.BlockSpec((_QT, _TOPK), lambda i, j: (i, 0)),
            pl.BlockSpec((_QT, _TOPK), lambda i, j: (i, 0)),
        ],
        out_shape=[
            jax.ShapeDtypeStruct((_Q, _TOPK), jnp.float32),
            jax.ShapeDtypeStruct((_Q, _TOPK), jnp.int32),
        ],
        scratch_shapes=[
            pltpu.VMEM((_QT, _KT), jnp.float32),
            pltpu.VMEM((_QT, _TOPK), jnp.float32),
            pltpu.VMEM((_QT, _TOPK), jnp.float32),
        ],
    )(queries, keys_p, ksq_p)
    return scores, idx
